# baseline (speedup 1.0000x reference)
import functools
import math

import jax
import jax.numpy as jnp
from jax import lax
from jax.experimental import pallas as pl
from jax.experimental.pallas import tpu as pltpu
from jax.experimental.pallas import tpu_sc as plsc

_VOCAB = 1000000
_EMB = 64
_B = 4096
_L = 200

_NC = 2
_NS = 16
_NW = _NC * _NS
_BLK = _B // _NW
_NBUF = 4
_NOB = 2
_SCALE = math.sqrt(_EMB)

_mesh = plsc.VectorSubcoreMesh(core_axis_name="c", subcore_axis_name="s")


@functools.partial(
    pl.kernel,
    mesh=_mesh,
    out_type=jax.ShapeDtypeStruct((_L, _EMB, _B), jnp.float32),
    scratch_types=(
        [pltpu.VMEM((_L, _BLK), jnp.int32)]
        + [pltpu.VMEM((_BLK,), jnp.int32) for _ in range(_NBUF)]
        + [pltpu.VMEM((_BLK, 2 * _EMB), jnp.float32) for _ in range(_NBUF)]
        + [pltpu.VMEM((_EMB, _BLK), jnp.float32) for _ in range(_NOB)]
        + [pltpu.SemaphoreType.DMA for _ in range(_NBUF + _NOB)]
    ),
    compiler_params=pltpu.CompilerParams(needs_layout_passes=False),
)
def _embed(tok_hbm, table_hbm, out_hbm, tokst, *refs):
    idxb = refs[:_NBUF]
    gbuf = refs[_NBUF:2 * _NBUF]
    obuf = refs[2 * _NBUF:2 * _NBUF + _NOB]
    gsem = refs[2 * _NBUF + _NOB:3 * _NBUF + _NOB]
    ssem = refs[3 * _NBUF + _NOB:3 * _NBUF + 2 * _NOB]

    wid = lax.axis_index("s") * _NC + lax.axis_index("c")
    b0 = wid * _BLK
    pltpu.sync_copy(tok_hbm.at[:, pl.ds(b0, _BLK)], tokst)

    def write_idx(l, b):
        for k in range(_BLK // 16):
            t = tokst[l, pl.ds(16 * k, 16)]
            idxb[b][pl.ds(16 * k, 16)] = lax.shift_right_logical(t, 1)

    def fire_gather(b):
        pltpu.async_copy(table_hbm.at[idxb[b]], gbuf[b], gsem[b])

    def wait_gather(b):
        pltpu.make_async_copy(table_hbm.at[idxb[b]], gbuf[b], gsem[b]).wait()

    def fire_store(l, ob):
        pltpu.async_copy(obuf[ob], out_hbm.at[l, :, pl.ds(b0, _BLK)], ssem[ob])

    def wait_store(l, ob):
        pltpu.make_async_copy(
            obuf[ob], out_hbm.at[l, :, pl.ds(b0, _BLK)], ssem[ob]).wait()

    def extract(l, b, ob):
        iota = lax.iota(jnp.int32, 16)
        rows = [iota + 16 * k for k in range(_BLK // 16)]
        cols = []
        for k in range(_BLK // 16):
            t = tokst[l, pl.ds(16 * k, 16)]
            cols.append(lax.shift_left(jnp.bitwise_and(t, 1), 6))

        def estep(e, carry):
            for k in range(_BLK // 16):
                v = plsc.load_gather(gbuf[b], [rows[k], cols[k] + e])
                obuf[ob][e, pl.ds(16 * k, 16)] = v * _SCALE
            return carry

        lax.fori_loop(0, _EMB, estep, None)

    for b in range(_NBUF - 1):
        write_idx(b, b)
        fire_gather(b)

    def outer(it, carry):
        l0 = it * _NBUF
        for b in range(_NBUF):
            l = l0 + b
            ob = b % _NOB
            wait_gather(b)

            @pl.when(l >= _NOB)
            def _():
                wait_store(l - _NOB, ob)

            extract(l, b, ob)
            fire_store(l, ob)

            @pl.when(l + _NBUF - 1 < _L)
            def _():
                bn = (b + _NBUF - 1) % _NBUF
                write_idx(l + _NBUF - 1, bn)
                fire_gather(bn)

        return carry

    lax.fori_loop(0, _L // _NBUF, outer, None)

    for l in (_L - 2, _L - 1):
        wait_store(l, l % _NOB)


def kernel(tokens, table):
    tok_t = tokens.astype(jnp.int32).T
    table2 = jnp.reshape(table, (_VOCAB // 2, 2 * _EMB))
    out = _embed(tok_t, table2)
    return out.transpose(2, 0, 1)


# speedup vs baseline: 1.7153x; 1.7153x over previous
import functools
import math

import jax
import jax.numpy as jnp
from jax import lax
from jax.experimental import pallas as pl
from jax.experimental.pallas import tpu as pltpu
from jax.experimental.pallas import tpu_sc as plsc

_VOCAB = 1000000
_EMB = 64
_B = 4096
_L = 200

_NC = 2
_NS = 16
_NW = _NC * _NS
_BLK = _B // _NW
_NBUF = 4
_NOB = 2
_SCALE = math.sqrt(_EMB)

_mesh = plsc.VectorSubcoreMesh(core_axis_name="c", subcore_axis_name="s")


@functools.partial(
    pl.kernel,
    mesh=_mesh,
    out_type=jax.ShapeDtypeStruct((_L, _EMB, _B), jnp.float32),
    scratch_types=(
        [pltpu.VMEM((_L, _BLK), jnp.int32)]
        + [pltpu.VMEM((_BLK,), jnp.int32) for _ in range(_NBUF)]
        + [pltpu.VMEM((_BLK, 2 * _EMB), jnp.float32) for _ in range(_NBUF)]
        + [pltpu.VMEM((_EMB, _BLK), jnp.float32) for _ in range(_NOB)]
        + [pltpu.SemaphoreType.DMA for _ in range(_NBUF + _NOB)]
    ),
    compiler_params=pltpu.CompilerParams(needs_layout_passes=False),
)
def _embed(tok_hbm, table_hbm, out_hbm, tokst, *refs):
    idxb = refs[:_NBUF]
    gbuf = refs[_NBUF:2 * _NBUF]
    obuf = refs[2 * _NBUF:2 * _NBUF + _NOB]
    gsem = refs[2 * _NBUF + _NOB:3 * _NBUF + _NOB]
    ssem = refs[3 * _NBUF + _NOB:3 * _NBUF + 2 * _NOB]

    wid = lax.axis_index("s") * _NC + lax.axis_index("c")
    b0 = wid * _BLK
    pltpu.sync_copy(tok_hbm.at[:, pl.ds(b0, _BLK)], tokst)

    def write_idx(l, b):
        for k in range(_BLK // 16):
            t = tokst[l, pl.ds(16 * k, 16)]
            idxb[b][pl.ds(16 * k, 16)] = lax.shift_right_logical(t, 1)

    def fire_gather(b):
        pltpu.async_copy(table_hbm.at[idxb[b]], gbuf[b], gsem[b])

    def wait_gather(b):
        pltpu.make_async_copy(table_hbm.at[idxb[b]], gbuf[b], gsem[b]).wait()

    def fire_store(l, ob):
        pltpu.async_copy(obuf[ob], out_hbm.at[l, :, pl.ds(b0, _BLK)], ssem[ob])

    def wait_store(l, ob):
        pltpu.make_async_copy(
            obuf[ob], out_hbm.at[l, :, pl.ds(b0, _BLK)], ssem[ob]).wait()

    def extract(l, b, ob):
        iota = lax.iota(jnp.int32, 16)
        rows = [iota + 16 * k for k in range(_BLK // 16)]
        par64 = []
        for k in range(_BLK // 16):
            t = tokst[l, pl.ds(16 * k, 16)]
            par64.append(lax.shift_left(jnp.bitwise_and(t, 1), 6))
        rot = [jnp.bitwise_and(iota + i, 15) for i in range(16)]

        def estep(eb, carry):
            e0 = eb * 16
            er = [rot[i] + e0 for i in range(16)]
            for k in range(_BLK // 16):
                for i in range(16):
                    v = plsc.load_gather(gbuf[b], [rows[k], par64[k] + er[i]])
                    plsc.store_scatter(obuf[ob], [er[i], rows[k]], v * _SCALE)
            return carry

        lax.fori_loop(0, _EMB // 16, estep, None)

    for b in range(_NBUF - 1):
        write_idx(b, b)
        fire_gather(b)

    def outer(it, carry):
        l0 = it * _NBUF
        for b in range(_NBUF):
            l = l0 + b
            ob = b % _NOB
            wait_gather(b)

            @pl.when(l >= _NOB)
            def _():
                wait_store(l - _NOB, ob)

            extract(l, b, ob)
            fire_store(l, ob)

            @pl.when(l + _NBUF - 1 < _L)
            def _():
                bn = (b + _NBUF - 1) % _NBUF
                write_idx(l + _NBUF - 1, bn)
                fire_gather(bn)

        return carry

    lax.fori_loop(0, _L // _NBUF, outer, None)

    for l in (_L - 2, _L - 1):
        wait_store(l, l % _NOB)


def kernel(tokens, table):
    tok_t = tokens.astype(jnp.int32).T
    table2 = jnp.reshape(table, (_VOCAB // 2, 2 * _EMB))
    out = _embed(tok_t, table2)
    return out.transpose(2, 0, 1)
